# Initial kernel scaffold; baseline (speedup 1.0000x reference)
import jax
import jax.numpy as jnp
from jax.experimental import pallas as pl


def kernel(inputs):
    raise NotImplementedError("write your pallas kernel here")


# speedup vs baseline: 1.6134x; 1.6134x over previous
import jax
import jax.numpy as jnp
from jax.experimental import pallas as pl
from jax.experimental.pallas import tpu as pltpu

B = 128
N = 8192
D = 16
FLAT = N * D
ROWS = FLAT // 128


def _mask_row_kernel(x_ref, o_ref):
    x = x_ref[0]
    sq = x * x
    kk = jax.lax.broadcasted_iota(jnp.int32, (128, 128), 0)
    ll = jax.lax.broadcasted_iota(jnp.int32, (128, 128), 1)
    g = ((kk // D) == (ll // D)).astype(jnp.float32)
    t = jax.lax.dot(sq, g, precision=jax.lax.Precision.HIGHEST,
                    preferred_element_type=jnp.float32)
    norm = jnp.sqrt(t)
    m = jnp.max(norm)
    flatpos = (jax.lax.broadcasted_iota(jnp.int32, (ROWS, 128), 0) * 128
               + jax.lax.broadcasted_iota(jnp.int32, (ROWS, 128), 1))
    fmin = jnp.min(jnp.where(norm == m, flatpos, FLAT))
    keep = (flatpos >= fmin) & (flatpos < fmin + D)
    o_ref[0] = jnp.where(keep, x, 0.0)


def kernel(inputs):
    flat = inputs.reshape(B, ROWS, 128)
    out = pl.pallas_call(
        _mask_row_kernel,
        grid=(B,),
        in_specs=[pl.BlockSpec((1, ROWS, 128), lambda b: (b, 0, 0))],
        out_specs=pl.BlockSpec((1, ROWS, 128), lambda b: (b, 0, 0)),
        out_shape=jax.ShapeDtypeStruct((B, ROWS, 128), jnp.float32),
    )(flat)
    return out.reshape(B, FLAT)
